# baseline (speedup 1.0000x reference)
import functools

import jax
import jax.numpy as jnp
from jax import lax
from jax.experimental import pallas as pl
from jax.experimental.pallas import tpu as pltpu
from jax.experimental.pallas import tpu_sc as plsc

B = 16384
EMBED = 64
FEAT = 16
N_USERS = 1000000
NC = 2
NS = 16
NW = NC * NS
BPW = B // NW
CH = 32
NCH = BPW // CH
T_MAX = N_USERS // 128 - 1
U_TAIL = (T_MAX + 1) * 128
BT = 2
BW_ = BT * 128
B0_MAX = (N_USERS - BW_) // 128


def _scan_body(uid_hbm, utT_hbm, ueT_hbm, idx_u, bbuf, out_u, usem):
    c = lax.axis_index("c")
    s = lax.axis_index("s")
    wid = s * NC + c
    base = wid * BPW
    pltpu.sync_copy(uid_hbm.at[pl.ds(base, BPW)], idx_u.at[pl.ds(0, BPW)])
    idx_u[pl.ds(BPW, 16)] = jnp.full((16,), jnp.int32(0x7FFFFFF))

    def tile_of(j):
        v = idx_u[pl.ds(j, 16)]
        return jnp.minimum(lax.shift_right_logical(v[0], 7), T_MAX)

    t0 = tile_of(0)
    t1 = tile_of(BPW - 16 + 15)
    nb = lax.shift_right_logical(t1 - t0 + BT, BT.bit_length() - 1)

    def batch_copy(q, p):
        bt = jnp.minimum(t0 + q * BT, B0_MAX)
        off = pl.multiple_of(bt * 128, 128)
        return pltpu.make_async_copy(
            utT_hbm.at[:, pl.ds(off, BW_)], bbuf.at[p], usem)

    batch_copy(0, 0).start()

    @pl.when(nb > 1)
    def _():
        batch_copy(1, 1).start()

    @pl.when(nb > 2)
    def _():
        batch_copy(2, 2).start()

    batch_copy(0, 0).wait()

    lanes = lax.iota(jnp.int32, 16)

    def step(i, state):
        q, j = state
        b_lo = jnp.minimum(t0 + q * BT, B0_MAX) * 128
        b_hi = b_lo + BW_
        v = idx_u[pl.ds(j, 16)]
        last = q >= nb - 1
        in_hi = jnp.logical_or(v < b_hi, last)
        m = jnp.logical_and(
            jnp.logical_and(v >= b_lo, in_hi),
            lanes + j < BPW)
        cnt = plsc.all_reduce_population_count(m)[0]
        adv = jnp.logical_and(cnt == 0, jnp.logical_not(last))

        @pl.when(adv)
        def _():
            @pl.when(q + 3 < nb)
            def _():
                batch_copy(q + 3, lax.rem(q, 3)).start()

            batch_copy(q + 1, lax.rem(q + 1, 3)).wait()

        @pl.when(cnt > 0)
        def _():
            p = lax.rem(q, 3)
            col = jnp.clip(v - b_lo, 0, BW_ - 1)
            dst = lanes + j
            for e in range(EMBED):
                ev = jnp.full((16,), jnp.int32(e))
                vals = plsc.load_gather(bbuf.at[p], [ev, col])
                plsc.store_scatter(out_u, [ev, dst], vals)

        q2 = jnp.where(adv, q + 1, q)
        j2 = jnp.where(adv, j, j + cnt)
        return (q2, j2)

    lax.fori_loop(0, nb + BPW, step, (jnp.int32(0), jnp.int32(0)),
                  unroll=False)
    pltpu.sync_copy(out_u.at[:, pl.ds(0, BPW)],
                    ueT_hbm.at[:, pl.ds(base, BPW)])


@functools.cache
def _scan():
    return pl.kernel(
        _scan_body,
        out_type=jax.ShapeDtypeStruct((EMBED, B), jnp.float32),
        mesh=plsc.VectorSubcoreMesh(
            core_axis_name="c", subcore_axis_name="s",
            num_cores=NC, num_subcores=NS),
        scratch_types=[
            pltpu.VMEM((BPW + 16,), jnp.int32),
            pltpu.VMEM((3, EMBED, BW_), jnp.float32),
            pltpu.VMEM((EMBED, BPW + 128), jnp.float32),
            pltpu.SemaphoreType.DMA,
        ],
        compiler_params=pltpu.CompilerParams(
            use_tc_tiling_on_sc=True, needs_layout_passes=False),
    )


def _rows_body(iid_hbm, it_hbm, ft_hbm, ieT_hbm, feT_hbm,
               idx_i, buf_i, buf_f, obuf_i, obuf_f, sem):
    c = lax.axis_index("c")
    s = lax.axis_index("s")
    wid = s * NC + c
    base = wid * BPW
    pltpu.sync_copy(iid_hbm.at[pl.ds(base, BPW)], idx_i)
    lanes = lax.iota(jnp.int32, 16)

    def fire(k):
        p = lax.rem(k, 2)
        descs = []
        for g in range(CH // 16):
            ivec = idx_i[pl.ds(k * CH + g * 16, 16)]
            for l in range(16):
                r = g * 16 + l
                i = ivec[l]
                descs.append(pltpu.async_copy(
                    it_hbm.at[pl.ds(i, 1), :],
                    buf_i.at[p, pl.ds(r, 1), :], sem))
                descs.append(pltpu.async_copy(
                    ft_hbm.at[pl.ds(i, 1), :],
                    buf_f.at[p, pl.ds(r, 1), :], sem))
        return descs

    def extract(k):
        p = lax.rem(k, 2)
        for r in range(CH):
            col = jnp.full((16,), k * CH + r)
            for e4 in range(EMBED // 16):
                vals = buf_i[p, r, pl.ds(16 * e4, 16)]
                plsc.store_scatter(obuf_i, [lanes + 16 * e4, col], vals)
            fvals = buf_f[p, r, pl.ds(0, FEAT)]
            plsc.store_scatter(obuf_f, [lanes, col], fvals)

    def chunk(k, _):
        descs = fire(k)

        @pl.when(k >= 1)
        def _():
            extract(k - 1)
        for d in descs:
            d.wait()
        return ()

    lax.fori_loop(0, NCH, chunk, (), unroll=False)
    extract(NCH - 1)
    pltpu.sync_copy(obuf_i, ieT_hbm.at[:, pl.ds(base, BPW)])
    pltpu.sync_copy(obuf_f, feT_hbm.at[:, pl.ds(base, BPW)])


@functools.cache
def _rows():
    return pl.kernel(
        _rows_body,
        out_type=(
            jax.ShapeDtypeStruct((EMBED, B), jnp.float32),
            jax.ShapeDtypeStruct((FEAT, B), jnp.float32),
        ),
        mesh=plsc.VectorSubcoreMesh(
            core_axis_name="c", subcore_axis_name="s",
            num_cores=NC, num_subcores=NS),
        scratch_types=[
            pltpu.VMEM((BPW,), jnp.int32),
            pltpu.VMEM((2, CH, EMBED), jnp.float32),
            pltpu.VMEM((2, CH, FEAT), jnp.float32),
            pltpu.VMEM((EMBED, BPW), jnp.float32),
            pltpu.VMEM((FEAT, BPW), jnp.float32),
            pltpu.SemaphoreType.DMA,
        ],
        compiler_params=pltpu.CompilerParams(
            use_tc_tiling_on_sc=True, needs_layout_passes=False),
    )


BLK = 2048


def _mlp_body(uid, ueT, ieT, feT, tail, W1T, b1, W2T, b2, W3T, b3, WwT, bw,
              out):
    uid_r = uid[...]
    ueT_v = ueT[...]
    ieT_v = ieT[...]
    feT_v = feT[...]
    tail_off = uid_r - U_TAIL
    rows = lax.broadcasted_iota(jnp.int32, (EMBED, BLK), 0)
    ohT = (rows == tail_off).astype(jnp.float32)
    dg = lambda a, b: lax.dot_general(
        a, b, (((0,), (0,)), ((), ())),
        preferred_element_type=jnp.float32)
    corrT = dg(tail[...], ohT)
    ueT_u = jnp.where(uid_r >= U_TAIL, corrT, ueT_v)

    dot = functools.partial(jnp.dot, preferred_element_type=jnp.float32)
    W1T_v = W1T[...]
    WwT_v = WwT[...]
    h1 = dot(W1T_v[:, :EMBED], ueT_u) + dot(W1T_v[:, EMBED:2 * EMBED], ieT_v)
    h1 = h1 + dot(W1T_v[:, 2 * EMBED:], feT_v) + b1[...]
    h1 = jnp.maximum(h1, 0.0)
    h2 = jnp.maximum(dot(W2T[...], h1) + b2[...], 0.0)
    deep = jnp.maximum(dot(W3T[...], h2) + b3[...], 0.0)
    wide = (dot(WwT_v[:, :EMBED], ueT_u) + dot(WwT_v[:, EMBED:2 * EMBED], ieT_v)
            + dot(WwT_v[:, 2 * EMBED:], feT_v) + bw[...])
    out[...] = (deep + wide)[0, :]


def _mlp(uid_s, ueT, ieT, feT, tail, W1T, b1, W2T, b2, W3T, b3, WwT, bw):
    d_in = 2 * EMBED + FEAT
    grid = B // BLK
    cols = lambda i: (0, i)
    full = lambda i: (0, 0)
    return pl.pallas_call(
        _mlp_body,
        grid=(grid,),
        in_specs=[
            pl.BlockSpec((1, BLK), cols),
            pl.BlockSpec((EMBED, BLK), cols),
            pl.BlockSpec((EMBED, BLK), cols),
            pl.BlockSpec((FEAT, BLK), cols),
            pl.BlockSpec((EMBED, EMBED), full),
            pl.BlockSpec((64, d_in), full),
            pl.BlockSpec((64, 1), full),
            pl.BlockSpec((32, 64), full),
            pl.BlockSpec((32, 1), full),
            pl.BlockSpec((1, 32), full),
            pl.BlockSpec((1, 1), full),
            pl.BlockSpec((1, d_in), full),
            pl.BlockSpec((1, 1), full),
        ],
        out_specs=pl.BlockSpec((BLK,), lambda i: (i,)),
        out_shape=jax.ShapeDtypeStruct((B,), jnp.float32),
    )(uid_s, ueT, ieT, feT, tail, W1T, b1, W2T, b2, W3T, b3, WwT, bw)


def kernel(user_id, item_id, user_table, item_table, feat_table,
           W1, b1, W2, b2, W3, b3, Ww, bw):
    uid = user_id.astype(jnp.int32)
    iid = item_id.astype(jnp.int32)
    pos = lax.iota(jnp.int32, B)
    uid_s, iid_s, perm = lax.sort((uid, iid, pos), dimension=0, num_keys=1)
    tail = user_table[U_TAIL:, :]
    ueT_s = _scan()(uid_s, user_table.T)
    ieT_s, feT_s = _rows()(iid_s, item_table, feat_table)
    res_s = _mlp(uid_s.reshape(1, B), ueT_s, ieT_s, feT_s, tail,
                 W1.T, b1.reshape(64, 1), W2.T, b2.reshape(32, 1),
                 W3.T, b3.reshape(1, 1), Ww.T, bw.reshape(1, 1))
    return lax.sort((perm, res_s), dimension=0, num_keys=1)[1]


# speedup vs baseline: 1.0527x; 1.0527x over previous
import functools

import jax
import jax.numpy as jnp
from jax import lax
from jax.experimental import pallas as pl
from jax.experimental.pallas import tpu as pltpu
from jax.experimental.pallas import tpu_sc as plsc

B = 16384
EMBED = 64
FEAT = 16
N_USERS = 1000000
NC = 2
NS = 16
NW = NC * NS
BPW = B // NW
CH = 64
NCH = BPW // CH
T_MAX = N_USERS // 128 - 1
U_TAIL = (T_MAX + 1) * 128
BT = 2
BW_ = BT * 128
B0_MAX = (N_USERS - BW_) // 128


def _scan_body(uid_hbm, utT_hbm, ueT_hbm, idx_u, bbuf, out_u, usem):
    c = lax.axis_index("c")
    s = lax.axis_index("s")
    wid = s * NC + c
    base = wid * BPW
    pltpu.sync_copy(uid_hbm.at[pl.ds(base, BPW)], idx_u.at[pl.ds(0, BPW)])
    idx_u[pl.ds(BPW, 16)] = jnp.full((16,), jnp.int32(0x7FFFFFF))

    def tile_of(j):
        v = idx_u[pl.ds(j, 16)]
        return jnp.minimum(lax.shift_right_logical(v[0], 7), T_MAX)

    t0 = tile_of(0)
    t1 = tile_of(BPW - 16 + 15)
    nb = lax.shift_right_logical(t1 - t0 + BT, BT.bit_length() - 1)

    def batch_copy(q, p):
        bt = jnp.minimum(t0 + q * BT, B0_MAX)
        off = pl.multiple_of(bt * 128, 128)
        return pltpu.make_async_copy(
            utT_hbm.at[:, pl.ds(off, BW_)], bbuf.at[p], usem)

    batch_copy(0, 0).start()

    @pl.when(nb > 1)
    def _():
        batch_copy(1, 1).start()

    @pl.when(nb > 2)
    def _():
        batch_copy(2, 2).start()

    batch_copy(0, 0).wait()

    lanes = lax.iota(jnp.int32, 16)

    def step(i, state):
        q, j = state
        b_lo = jnp.minimum(t0 + q * BT, B0_MAX) * 128
        b_hi = b_lo + BW_
        v = idx_u[pl.ds(j, 16)]
        last = q >= nb - 1
        in_hi = jnp.logical_or(v < b_hi, last)
        m = jnp.logical_and(
            jnp.logical_and(v >= b_lo, in_hi),
            lanes + j < BPW)
        cnt = plsc.all_reduce_population_count(m)[0]
        adv = jnp.logical_and(cnt == 0, jnp.logical_not(last))

        @pl.when(adv)
        def _():
            @pl.when(q + 3 < nb)
            def _():
                batch_copy(q + 3, lax.rem(q, 3)).start()

            batch_copy(q + 1, lax.rem(q + 1, 3)).wait()

        @pl.when(cnt > 0)
        def _():
            p = lax.rem(q, 3)
            col = jnp.clip(v - b_lo, 0, BW_ - 1)
            dst = lanes + j
            for e in range(EMBED):
                ev = jnp.full((16,), jnp.int32(e))
                vals = plsc.load_gather(bbuf.at[p], [ev, col])
                plsc.store_scatter(out_u, [ev, dst], vals)

        q2 = jnp.where(adv, q + 1, q)
        j2 = jnp.where(adv, j, j + cnt)
        return (q2, j2)

    lax.fori_loop(0, nb + BPW, step, (jnp.int32(0), jnp.int32(0)),
                  unroll=False)
    pltpu.sync_copy(out_u.at[:, pl.ds(0, BPW)],
                    ueT_hbm.at[:, pl.ds(base, BPW)])


@functools.cache
def _scan():
    return pl.kernel(
        _scan_body,
        out_type=jax.ShapeDtypeStruct((EMBED, B), jnp.float32),
        mesh=plsc.VectorSubcoreMesh(
            core_axis_name="c", subcore_axis_name="s",
            num_cores=NC, num_subcores=NS),
        scratch_types=[
            pltpu.VMEM((BPW + 16,), jnp.int32),
            pltpu.VMEM((3, EMBED, BW_), jnp.float32),
            pltpu.VMEM((EMBED, BPW + 128), jnp.float32),
            pltpu.SemaphoreType.DMA,
        ],
        compiler_params=pltpu.CompilerParams(
            use_tc_tiling_on_sc=True, needs_layout_passes=False),
    )


def _rows_body(iid_hbm, it_hbm, fp_hbm, ieT_hbm, feT_hbm,
               idx_i, buf_i, buf_f, obuf_i, obuf_f, sem):
    c = lax.axis_index("c")
    s = lax.axis_index("s")
    wid = s * NC + c
    base = wid * BPW
    pltpu.sync_copy(iid_hbm.at[pl.ds(base, BPW)], idx_i)
    lanes = lax.iota(jnp.int32, 16)

    def chunk(k, _):
        p = lax.rem(k, 2)
        descs = []
        for g in range(CH // 16):
            ivec = idx_i[pl.ds(k * CH + g * 16, 16)]
            for l in range(16):
                r = g * 16 + l
                i = ivec[l]
                descs.append(pltpu.async_copy(
                    it_hbm.at[pl.ds(i, 1), :],
                    buf_i.at[p, pl.ds(r, 1), :], sem))
                descs.append(pltpu.async_copy(
                    fp_hbm.at[pl.ds(lax.shift_right_logical(i, 3), 1), :],
                    buf_f.at[p, pl.ds(r, 1), :], sem))
        for d in descs:
            d.wait()
        for g in range(CH // 16):
            ivec = idx_i[pl.ds(k * CH + g * 16, 16)]
            for l in range(16):
                r = g * 16 + l
                col = jnp.full((16,), k * CH + r)
                for e4 in range(EMBED // 16):
                    vals = buf_i[p, r, pl.ds(16 * e4, 16)]
                    plsc.store_scatter(obuf_i, [lanes + 16 * e4, col], vals)
                foff = lax.bitwise_and(ivec[l], 7) * 16
                fvals = buf_f[p, r, pl.ds(foff, 16)]
                plsc.store_scatter(obuf_f, [lanes, col], fvals)
        return ()

    lax.fori_loop(0, NCH, chunk, (), unroll=False)
    pltpu.sync_copy(obuf_i, ieT_hbm.at[:, pl.ds(base, BPW)])
    pltpu.sync_copy(obuf_f, feT_hbm.at[:, pl.ds(base, BPW)])


@functools.cache
def _rows():
    return pl.kernel(
        _rows_body,
        out_type=(
            jax.ShapeDtypeStruct((EMBED, B), jnp.float32),
            jax.ShapeDtypeStruct((FEAT, B), jnp.float32),
        ),
        mesh=plsc.VectorSubcoreMesh(
            core_axis_name="c", subcore_axis_name="s",
            num_cores=NC, num_subcores=NS),
        scratch_types=[
            pltpu.VMEM((BPW,), jnp.int32),
            pltpu.VMEM((2, CH, EMBED), jnp.float32),
            pltpu.VMEM((2, CH, 128), jnp.float32),
            pltpu.VMEM((EMBED, BPW), jnp.float32),
            pltpu.VMEM((FEAT, BPW), jnp.float32),
            pltpu.SemaphoreType.DMA,
        ],
        compiler_params=pltpu.CompilerParams(
            use_tc_tiling_on_sc=True, needs_layout_passes=False),
    )


BLK = 2048


def _mlp_body(uid, ueT, ieT, feT, tail, W1T, b1, W2T, b2, W3T, b3, WwT, bw,
              out):
    uid_r = uid[...]
    ueT_v = ueT[...]
    ieT_v = ieT[...]
    feT_v = feT[...]
    tail_off = uid_r - U_TAIL
    rows = lax.broadcasted_iota(jnp.int32, (EMBED, BLK), 0)
    ohT = (rows == tail_off).astype(jnp.float32)
    dg = lambda a, b: lax.dot_general(
        a, b, (((0,), (0,)), ((), ())),
        preferred_element_type=jnp.float32)
    corrT = dg(tail[...], ohT)
    ueT_u = jnp.where(uid_r >= U_TAIL, corrT, ueT_v)

    dot = functools.partial(jnp.dot, preferred_element_type=jnp.float32)
    W1T_v = W1T[...]
    WwT_v = WwT[...]
    h1 = dot(W1T_v[:, :EMBED], ueT_u) + dot(W1T_v[:, EMBED:2 * EMBED], ieT_v)
    h1 = h1 + dot(W1T_v[:, 2 * EMBED:], feT_v) + b1[...]
    h1 = jnp.maximum(h1, 0.0)
    h2 = jnp.maximum(dot(W2T[...], h1) + b2[...], 0.0)
    deep = jnp.maximum(dot(W3T[...], h2) + b3[...], 0.0)
    wide = (dot(WwT_v[:, :EMBED], ueT_u) + dot(WwT_v[:, EMBED:2 * EMBED], ieT_v)
            + dot(WwT_v[:, 2 * EMBED:], feT_v) + bw[...])
    out[...] = (deep + wide)[0, :]


def _mlp(uid_s, ueT, ieT, feT, tail, W1T, b1, W2T, b2, W3T, b3, WwT, bw):
    d_in = 2 * EMBED + FEAT
    grid = B // BLK
    cols = lambda i: (0, i)
    full = lambda i: (0, 0)
    return pl.pallas_call(
        _mlp_body,
        grid=(grid,),
        in_specs=[
            pl.BlockSpec((1, BLK), cols),
            pl.BlockSpec((EMBED, BLK), cols),
            pl.BlockSpec((EMBED, BLK), cols),
            pl.BlockSpec((FEAT, BLK), cols),
            pl.BlockSpec((EMBED, EMBED), full),
            pl.BlockSpec((64, d_in), full),
            pl.BlockSpec((64, 1), full),
            pl.BlockSpec((32, 64), full),
            pl.BlockSpec((32, 1), full),
            pl.BlockSpec((1, 32), full),
            pl.BlockSpec((1, 1), full),
            pl.BlockSpec((1, d_in), full),
            pl.BlockSpec((1, 1), full),
        ],
        out_specs=pl.BlockSpec((BLK,), lambda i: (i,)),
        out_shape=jax.ShapeDtypeStruct((B,), jnp.float32),
    )(uid_s, ueT, ieT, feT, tail, W1T, b1, W2T, b2, W3T, b3, WwT, bw)


def kernel(user_id, item_id, user_table, item_table, feat_table,
           W1, b1, W2, b2, W3, b3, Ww, bw):
    uid = user_id.astype(jnp.int32)
    iid = item_id.astype(jnp.int32)
    pos = lax.iota(jnp.int32, B)
    uid_s, iid_s, perm = lax.sort((uid, iid, pos), dimension=0, num_keys=1)
    tail = user_table[U_TAIL:, :]
    n_items = feat_table.shape[0]
    featp = feat_table.reshape(n_items * FEAT // 128, 128)
    ueT_s = _scan()(uid_s, user_table.T)
    ieT_s, feT_s = _rows()(iid_s, item_table, featp)
    res_s = _mlp(uid_s.reshape(1, B), ueT_s, ieT_s, feT_s, tail,
                 W1.T, b1.reshape(64, 1), W2.T, b2.reshape(32, 1),
                 W3.T, b3.reshape(1, 1), Ww.T, bw.reshape(1, 1))
    return lax.sort((perm, res_s), dimension=0, num_keys=1)[1]
